# Initial kernel scaffold; baseline (speedup 1.0000x reference)
import jax
import jax.numpy as jnp
from jax.experimental import pallas as pl


def kernel(x, W_qkv, b_qkv):
    raise NotImplementedError("write your pallas kernel here")


# speedup vs baseline: 14.8297x; 14.8297x over previous
import functools

import jax
import jax.numpy as jnp
from jax import lax
from jax.experimental import pallas as pl
from jax.experimental.pallas import tpu as pltpu
from jax.experimental.pallas import tpu_sc as plsc

DIM = 768
NUM_HEADS = 8
TOPK = 4
HEAD_DIM = DIM // NUM_HEADS
SEQ = 1024

SC_CORES = 2
SC_SUBCORES = 16
SC_WORKERS = SC_CORES * SC_SUBCORES
SC_LANES = 16
CHUNK = 32
VPAD = 128


MATMUL_PRECISION = jax.lax.Precision.DEFAULT


def _tc_body(x_ref, wt_ref, br_ref, q_ref, v_ref, w_ref, i_ref):
    n = SEQ
    d = HEAD_DIM
    xb = x_ref[0]
    hi = jnp.float32(jnp.finfo(jnp.float32).max)

    def proj(t):
        wm = wt_ref[t, 0]
        bb = br_ref[t, 0]
        return (
            jax.lax.dot_general(
                xb, wm, (((1,), (0,)), ((), ())),
                preferred_element_type=jnp.float32,
                precision=MATMUL_PRECISION,
            )
            + bb
        )

    q = proj(0)
    k = proj(1)
    v = proj(2)
    attn = jax.lax.dot_general(
        q, k, (((1,), (1,)), ((), ())),
        preferred_element_type=jnp.float32,
        precision=MATMUL_PRECISION,
    )

    nck = n // 128
    iota128 = jax.lax.broadcasted_iota(jnp.int32, (1, 128), 1).astype(jnp.float32)
    a = attn
    vals = []
    idxs = []
    for p in range(TOPK):
        m = jnp.max(a, axis=1)
        ef = (a == m[:, None]).astype(jnp.float32)
        chunks = [ef[:, 128 * c : 128 * (c + 1)] for c in range(nck)]
        s_lane = chunks[0]
        cw = jnp.zeros_like(chunks[0])
        for c in range(1, nck):
            s_lane = s_lane + chunks[c]
            cw = cw + jnp.float32(c) * chunks[c]
        lane = jnp.sum(s_lane * iota128, axis=1)
        ck = jnp.sum(cw, axis=1)
        vals.append(m)
        idxs.append(ck * 128 + lane)
        if p < TOPK - 1:
            a = a - ef * hi

    m0 = vals[0]
    exps = [jnp.exp(vv - m0) for vv in vals]
    s = exps[0] + exps[1] + exps[2] + exps[3]
    w = jnp.stack([ee / s for ee in exps], axis=1)
    base = (pl.program_id(0) * NUM_HEADS + pl.program_id(1)) * n
    ig = jnp.stack(idxs, axis=1).astype(jnp.int32) + base

    q_ref[0, 0] = q
    v_ref[0, 0] = jnp.concatenate(
        [v, jnp.zeros((n, VPAD - d), jnp.float32)], axis=1
    )
    w_ref[0, 0] = w
    i_ref[0, 0] = ig


def _tc_stage(x_r, wt, br, batch):
    n, d, h = SEQ, HEAD_DIM, NUM_HEADS
    c = DIM
    grid = (batch, h)
    return pl.pallas_call(
        _tc_body,
        grid=grid,
        in_specs=[
            pl.BlockSpec((1, n, c), lambda b, hh: (b, 0, 0)),
            pl.BlockSpec((3, 1, c, d), lambda b, hh: (0, hh, 0, 0)),
            pl.BlockSpec((3, 1, 1, d), lambda b, hh: (0, hh, 0, 0)),
        ],
        out_specs=[
            pl.BlockSpec((1, 1, n, d), lambda b, hh: (b, hh, 0, 0)),
            pl.BlockSpec((1, 1, n, VPAD), lambda b, hh: (b, hh, 0, 0)),
            pl.BlockSpec((1, 1, n, TOPK), lambda b, hh: (b, hh, 0, 0)),
            pl.BlockSpec((1, 1, n, TOPK), lambda b, hh: (b, hh, 0, 0)),
        ],
        out_shape=[
            jax.ShapeDtypeStruct((batch, h, n, d), jnp.float32),
            jax.ShapeDtypeStruct((batch, h, n, VPAD), jnp.float32),
            jax.ShapeDtypeStruct((batch, h, n, TOPK), jnp.float32),
            jax.ShapeDtypeStruct((batch, h, n, TOPK), jnp.int32),
        ],
    )(x_r, wt, br)


def _sc_combine(v_flat, q_flat, w_flat, i_flat):
    rows = v_flat.shape[0]
    d = HEAD_DIM
    rows_per_w = rows // SC_WORKERS
    nchunk = rows_per_w // CHUNK
    mesh = plsc.VectorSubcoreMesh(
        core_axis_name="c", subcore_axis_name="s",
        num_cores=SC_CORES, num_subcores=SC_SUBCORES,
    )

    @functools.partial(
        pl.kernel,
        mesh=mesh,
        out_type=jax.ShapeDtypeStruct((rows, d), jnp.float32),
        scratch_types=[
            pltpu.VMEM((CHUNK * TOPK,), jnp.int32),
            pltpu.VMEM((CHUNK * TOPK + SC_LANES,), jnp.float32),
            pltpu.VMEM((CHUNK * TOPK, VPAD), jnp.float32),
            pltpu.VMEM((CHUNK, d), jnp.float32),
            pltpu.VMEM((CHUNK, d), jnp.float32),
            pltpu.SemaphoreType.DMA,
        ],
    )
    def sc_k(v_hbm, q_hbm, w_hbm, i_hbm, out_hbm, idx_v, w_v, g_v, q_v, o_v, sem):
        wid = lax.axis_index("s") * SC_CORES + lax.axis_index("c")

        def chunk_body(ci, carry):
            rbase = wid * rows_per_w + ci * CHUNK
            pltpu.sync_copy(i_hbm.at[pl.ds(rbase * TOPK, CHUNK * TOPK)], idx_v)
            pltpu.sync_copy(
                w_hbm.at[pl.ds(rbase * TOPK, CHUNK * TOPK)],
                w_v.at[pl.ds(0, CHUNK * TOPK)],
            )
            pltpu.sync_copy(q_hbm.at[pl.ds(rbase, CHUNK)], q_v)
            pltpu.async_copy(v_hbm.at[idx_v], g_v, sem).wait()

            def row_body(r, carry2):
                rb = r * TOPK
                wvec = w_v[pl.ds(rb, SC_LANES)]
                w0, w1, w2, w3 = wvec[0], wvec[1], wvec[2], wvec[3]
                for sgm in range(d // SC_LANES):
                    sl = pl.ds(sgm * SC_LANES, SC_LANES)
                    acc = w0 * g_v[rb, sl]
                    acc = acc + w1 * g_v[rb + 1, sl]
                    acc = acc + w2 * g_v[rb + 2, sl]
                    acc = acc + w3 * g_v[rb + 3, sl]
                    o_v[r, sl] = acc * q_v[r, sl]
                return carry2

            lax.fori_loop(0, CHUNK, row_body, 0)
            pltpu.sync_copy(o_v, out_hbm.at[pl.ds(rbase, CHUNK)])
            return carry

        lax.fori_loop(0, nchunk, chunk_body, 0)

    return sc_k(v_flat, q_flat, w_flat, i_flat)


def kernel(x, W_qkv, b_qkv):
    batch, c, hh, ww = x.shape
    n = hh * ww
    h, d = NUM_HEADS, HEAD_DIM
    xt = jnp.transpose(x.reshape(batch, c, n), (0, 2, 1))
    wt = jnp.transpose(W_qkv.reshape(c, 3, h, d), (1, 2, 0, 3))
    br = b_qkv.reshape(3, h, d)[:, :, None, :]

    q4, v4, w4, i4 = _tc_stage(xt, wt, br, batch)

    rows = batch * h * n
    out_flat = _sc_combine(
        v4.reshape(rows, VPAD),
        q4.reshape(rows, d),
        w4.reshape(rows * TOPK),
        i4.reshape(rows * TOPK),
    )
    out = out_flat.reshape(batch, h, n, d)
    return jnp.transpose(out, (0, 2, 1, 3)).reshape(batch, c, hh, ww)


# speedup vs baseline: 18.1643x; 1.2249x over previous
import functools

import jax
import jax.numpy as jnp
from jax import lax
from jax.experimental import pallas as pl
from jax.experimental.pallas import tpu as pltpu
from jax.experimental.pallas import tpu_sc as plsc

DIM = 768
NUM_HEADS = 8
TOPK = 4
HEAD_DIM = DIM // NUM_HEADS
SEQ = 1024

SC_CORES = 2
SC_SUBCORES = 16
SC_WORKERS = SC_CORES * SC_SUBCORES
SC_LANES = 16
CHUNK = 64
VPAD = 128


MATMUL_PRECISION = jax.lax.Precision.DEFAULT


def _tc_body(x_ref, wt_ref, br_ref, q_ref, v_ref, w_ref, i_ref):
    n = SEQ
    d = HEAD_DIM
    xb = x_ref[0]
    hi = jnp.float32(jnp.finfo(jnp.float32).max)

    def proj(t):
        wm = wt_ref[t, 0]
        bb = br_ref[t, 0]
        return (
            jax.lax.dot_general(
                xb, wm, (((1,), (0,)), ((), ())),
                preferred_element_type=jnp.float32,
                precision=MATMUL_PRECISION,
            )
            + bb
        )

    q = proj(0)
    k = proj(1)
    v = proj(2)
    attn = jax.lax.dot_general(
        q, k, (((1,), (1,)), ((), ())),
        preferred_element_type=jnp.float32,
        precision=MATMUL_PRECISION,
    )

    nck = n // 128
    iota128 = jax.lax.broadcasted_iota(jnp.int32, (1, 128), 1).astype(jnp.float32)
    a = attn
    vals = []
    idxs = []
    for p in range(TOPK):
        m = jnp.max(a, axis=1)
        ef = (a == m[:, None]).astype(jnp.float32)
        chunks = [ef[:, 128 * c : 128 * (c + 1)] for c in range(nck)]
        s_lane = chunks[0]
        cw = jnp.zeros_like(chunks[0])
        for c in range(1, nck):
            s_lane = s_lane + chunks[c]
            cw = cw + jnp.float32(c) * chunks[c]
        lane = jnp.sum(s_lane * iota128, axis=1)
        ck = jnp.sum(cw, axis=1)
        vals.append(m)
        idxs.append(ck * 128 + lane)
        if p < TOPK - 1:
            a = a - ef * hi

    m0 = vals[0]
    exps = [jnp.exp(vv - m0) for vv in vals]
    s = exps[0] + exps[1] + exps[2] + exps[3]
    w = jnp.stack([ee / s for ee in exps], axis=1)
    base = (pl.program_id(0) * NUM_HEADS + pl.program_id(1)) * n
    ig = jnp.stack(idxs, axis=1).astype(jnp.int32) + base

    pad = jnp.zeros((n, VPAD - d), jnp.float32)
    q_ref[0, 0] = jnp.concatenate([q, pad], axis=1)
    v_ref[0, 0] = jnp.concatenate([v, pad], axis=1)
    w_ref[0, 0] = w
    i_ref[0, 0] = ig


def _tc_stage(x_r, wt, br, batch):
    n, d, h = SEQ, HEAD_DIM, NUM_HEADS
    c = DIM
    grid = (batch, h)
    return pl.pallas_call(
        _tc_body,
        grid=grid,
        in_specs=[
            pl.BlockSpec((1, n, c), lambda b, hh: (b, 0, 0)),
            pl.BlockSpec((3, 1, c, d), lambda b, hh: (0, hh, 0, 0)),
            pl.BlockSpec((3, 1, 1, d), lambda b, hh: (0, hh, 0, 0)),
        ],
        out_specs=[
            pl.BlockSpec((1, 1, n, VPAD), lambda b, hh: (b, hh, 0, 0)),
            pl.BlockSpec((1, 1, n, VPAD), lambda b, hh: (b, hh, 0, 0)),
            pl.BlockSpec((1, 1, n, TOPK), lambda b, hh: (b, hh, 0, 0)),
            pl.BlockSpec((1, 1, n, TOPK), lambda b, hh: (b, hh, 0, 0)),
        ],
        out_shape=[
            jax.ShapeDtypeStruct((batch, h, n, VPAD), jnp.float32),
            jax.ShapeDtypeStruct((batch, h, n, VPAD), jnp.float32),
            jax.ShapeDtypeStruct((batch, h, n, TOPK), jnp.float32),
            jax.ShapeDtypeStruct((batch, h, n, TOPK), jnp.int32),
        ],
    )(x_r, wt, br)


def _sc_combine(v_flat, q_flat, w2, i2, batch):
    rows = v_flat.shape[0]
    d = HEAD_DIM
    rows_per_w = rows // SC_WORKERS
    nchunk = rows_per_w // CHUNK
    meta_rows = rows_per_w * TOPK // 128
    gpc = CHUNK * TOPK // 128
    mesh = plsc.VectorSubcoreMesh(
        core_axis_name="c", subcore_axis_name="s",
        num_cores=SC_CORES, num_subcores=SC_SUBCORES,
    )

    @functools.partial(
        pl.kernel,
        mesh=mesh,
        out_type=jax.ShapeDtypeStruct((rows, VPAD), jnp.float32),
        scratch_types=[
            pltpu.VMEM((meta_rows, 128), jnp.int32),
            pltpu.VMEM((rows_per_w * TOPK + SC_LANES,), jnp.float32),
            pltpu.VMEM((CHUNK * TOPK, VPAD), jnp.float32),
            pltpu.VMEM((CHUNK * TOPK, VPAD), jnp.float32),
            pltpu.VMEM((CHUNK, VPAD), jnp.float32),
            pltpu.VMEM((CHUNK, VPAD), jnp.float32),
            pltpu.VMEM((CHUNK, VPAD), jnp.float32),
            pltpu.SemaphoreType.DMA,
            pltpu.SemaphoreType.DMA,
        ],
    )
    def sc_k(v_hbm, q_hbm, w_hbm, i_hbm, out_hbm,
             idx_all, w_all, g0, g1, q0, q1, o_v, sem0, sem1):
        wid = lax.axis_index("s") * SC_CORES + lax.axis_index("c")
        wbase = wid * rows_per_w

        nmeta = rows_per_w * TOPK
        pltpu.sync_copy(i_hbm.at[pl.ds(wid * meta_rows, meta_rows)], idx_all)
        pltpu.sync_copy(
            w_hbm.at[pl.ds(wid * nmeta, nmeta)], w_all.at[pl.ds(0, nmeta)]
        )

        def q_copy(ci, q_v, sem):
            return pltpu.make_async_copy(
                q_hbm.at[pl.ds(wbase + ci * CHUNK, CHUNK)], q_v, sem
            )

        def g_copy(ci, g_v, k, sem):
            return pltpu.make_async_copy(
                v_hbm.at[idx_all.at[ci * gpc + k]],
                g_v.at[pl.ds(k * 128, 128)],
                sem,
            )

        def issue(ci, g_v, q_v, sem):
            q_copy(ci, q_v, sem).start()
            for k in range(gpc):
                g_copy(ci, g_v, k, sem).start()

        def drain(ci, g_v, q_v, sem):
            q_copy(ci, q_v, sem).wait()
            for k in range(gpc):
                g_copy(ci, g_v, k, sem).wait()

        issue(0, g0, q0, sem0)
        issue(1, g1, q1, sem1)

        def pair_body(i, carry):
            for sl_i, (g_v, q_v, sem) in enumerate(
                ((g0, q0, sem0), (g1, q1, sem1))
            ):
                ci = 2 * i + sl_i
                drain(ci, g_v, q_v, sem)

                def row_body(r, carry2):
                    wvec = w_all[pl.ds((ci * CHUNK + r) * TOPK, SC_LANES)]
                    w0, w1, w2, w3 = wvec[0], wvec[1], wvec[2], wvec[3]
                    rb = r * TOPK
                    for sgm in range(d // SC_LANES):
                        sl = pl.ds(sgm * SC_LANES, SC_LANES)
                        acc = w0 * g_v[rb, sl]
                        acc = acc + w1 * g_v[rb + 1, sl]
                        acc = acc + w2 * g_v[rb + 2, sl]
                        acc = acc + w3 * g_v[rb + 3, sl]
                        o_v[r, sl] = acc * q_v[r, sl]
                    return carry2

                lax.fori_loop(0, CHUNK, row_body, 0)
                pltpu.sync_copy(
                    o_v, out_hbm.at[pl.ds(wbase + ci * CHUNK, CHUNK)]
                )

                @pl.when(ci + 2 < nchunk)
                def _():
                    issue(ci + 2, g_v, q_v, sem)

            return carry

        lax.fori_loop(0, nchunk // 2, pair_body, 0)

    return sc_k(v_flat, q_flat, w2, i2)


def kernel(x, W_qkv, b_qkv):
    batch, c, hh, ww = x.shape
    n = hh * ww
    h, d = NUM_HEADS, HEAD_DIM
    xt = jnp.transpose(x.reshape(batch, c, n), (0, 2, 1))
    wt = jnp.transpose(W_qkv.reshape(c, 3, h, d), (1, 2, 0, 3))
    br = b_qkv.reshape(3, h, d)[:, :, None, :]

    q4, v4, w4, i4 = _tc_stage(xt, wt, br, batch)

    rows = batch * h * n
    out_p = _sc_combine(
        v4.reshape(rows, VPAD),
        q4.reshape(rows, VPAD),
        w4.reshape(rows * TOPK),
        i4.reshape(rows * TOPK // 128, 128),
        batch,
    )
    out = out_p.reshape(batch, h, n, VPAD)[..., :d]
    return jnp.transpose(out, (0, 2, 1, 3)).reshape(batch, c, hh, ww)


# speedup vs baseline: 19.2830x; 1.0616x over previous
import functools

import jax
import jax.numpy as jnp
from jax import lax
from jax.experimental import pallas as pl
from jax.experimental.pallas import tpu as pltpu
from jax.experimental.pallas import tpu_sc as plsc

DIM = 768
NUM_HEADS = 8
TOPK = 4
HEAD_DIM = DIM // NUM_HEADS
SEQ = 1024

SC_CORES = 2
SC_SUBCORES = 16
SC_WORKERS = SC_CORES * SC_SUBCORES
SC_LANES = 16
CHUNK = 64
VPAD = 128


MATMUL_PRECISION = jax.lax.Precision.DEFAULT


def _tc_body(x_ref, wt_ref, br_ref, q_ref, v_ref, w_ref, i_ref):
    n = SEQ
    d = HEAD_DIM
    xb = x_ref[0]
    hi = jnp.float32(jnp.finfo(jnp.float32).max)

    def proj(t):
        wm = wt_ref[t, 0]
        bb = br_ref[t, 0]
        return (
            jax.lax.dot_general(
                xb, wm, (((1,), (0,)), ((), ())),
                preferred_element_type=jnp.float32,
                precision=MATMUL_PRECISION,
            )
            + bb
        )

    q = proj(0)
    k = proj(1)
    v = proj(2)
    attn = jax.lax.dot_general(
        q, k, (((1,), (1,)), ((), ())),
        preferred_element_type=jnp.float32,
        precision=MATMUL_PRECISION,
    )

    nck = n // 128
    iota128 = jax.lax.broadcasted_iota(jnp.int32, (1, 128), 1).astype(jnp.float32)
    a = attn
    vals = []
    idxs = []
    for p in range(TOPK):
        m = jnp.max(a, axis=1)
        ef = (a == m[:, None]).astype(jnp.float32)
        chunks = [ef[:, 128 * c : 128 * (c + 1)] for c in range(nck)]
        s_lane = chunks[0]
        cw = jnp.zeros_like(chunks[0])
        for c in range(1, nck):
            s_lane = s_lane + chunks[c]
            cw = cw + jnp.float32(c) * chunks[c]
        lane = jnp.sum(s_lane * iota128, axis=1)
        ck = jnp.sum(cw, axis=1)
        vals.append(m)
        idxs.append(ck * 128 + lane)
        if p < TOPK - 1:
            a = a - ef * hi

    m0 = vals[0]
    exps = [jnp.exp(vv - m0) for vv in vals]
    s = exps[0] + exps[1] + exps[2] + exps[3]
    w = jnp.stack([ee / s for ee in exps], axis=1)
    base = (pl.program_id(0) * NUM_HEADS + pl.program_id(1)) * n
    ig = jnp.stack(idxs, axis=1).astype(jnp.int32) + base

    pad = jnp.zeros((n, VPAD - d), jnp.float32)
    q_ref[0, 0] = jnp.concatenate([q, pad], axis=1)
    v_ref[0, 0] = jnp.concatenate([v, pad], axis=1)
    w_ref[0, 0] = w
    i_ref[0, 0] = ig


def _tc_stage(x_r, wt, br, batch):
    n, d, h = SEQ, HEAD_DIM, NUM_HEADS
    c = DIM
    grid = (batch, h)
    return pl.pallas_call(
        _tc_body,
        grid=grid,
        in_specs=[
            pl.BlockSpec((1, n, c), lambda b, hh: (b, 0, 0)),
            pl.BlockSpec((3, 1, c, d), lambda b, hh: (0, hh, 0, 0)),
            pl.BlockSpec((3, 1, 1, d), lambda b, hh: (0, hh, 0, 0)),
        ],
        out_specs=[
            pl.BlockSpec((1, 1, n, VPAD), lambda b, hh: (b, hh, 0, 0)),
            pl.BlockSpec((1, 1, n, VPAD), lambda b, hh: (b, hh, 0, 0)),
            pl.BlockSpec((1, 1, n, TOPK), lambda b, hh: (b, hh, 0, 0)),
            pl.BlockSpec((1, 1, n, TOPK), lambda b, hh: (b, hh, 0, 0)),
        ],
        out_shape=[
            jax.ShapeDtypeStruct((batch, h, n, VPAD), jnp.float32),
            jax.ShapeDtypeStruct((batch, h, n, VPAD), jnp.float32),
            jax.ShapeDtypeStruct((batch, h, n, TOPK), jnp.float32),
            jax.ShapeDtypeStruct((batch, h, n, TOPK), jnp.int32),
        ],
    )(x_r, wt, br)


def _sc_combine(v_flat, q_flat, w2, i2, batch):
    rows = v_flat.shape[0]
    d = HEAD_DIM
    rows_per_w = rows // SC_WORKERS
    nchunk = rows_per_w // CHUNK
    meta_rows = rows_per_w * TOPK // 128
    gpc = CHUNK * TOPK // 128
    mesh = plsc.VectorSubcoreMesh(
        core_axis_name="c", subcore_axis_name="s",
        num_cores=SC_CORES, num_subcores=SC_SUBCORES,
    )

    @functools.partial(
        pl.kernel,
        mesh=mesh,
        out_type=jax.ShapeDtypeStruct((rows, VPAD), jnp.float32),
        scratch_types=[
            pltpu.VMEM((meta_rows, 128), jnp.int32),
            pltpu.VMEM((rows_per_w * TOPK + SC_LANES,), jnp.float32),
            pltpu.VMEM((CHUNK * TOPK, VPAD), jnp.float32),
            pltpu.VMEM((CHUNK * TOPK, VPAD), jnp.float32),
            pltpu.VMEM((CHUNK, VPAD), jnp.float32),
            pltpu.VMEM((CHUNK, VPAD), jnp.float32),
            pltpu.VMEM((CHUNK, VPAD), jnp.float32),
            pltpu.SemaphoreType.DMA,
            pltpu.SemaphoreType.DMA,
        ],
    )
    def sc_k(v_hbm, q_hbm, w_hbm, i_hbm, out_hbm,
             idx_all, w_all, g0, g1, q0, q1, o_v, sem0, sem1):
        wid = lax.axis_index("s") * SC_CORES + lax.axis_index("c")
        wbase = wid * rows_per_w

        nmeta = rows_per_w * TOPK
        pltpu.sync_copy(i_hbm.at[pl.ds(wid * meta_rows, meta_rows)], idx_all)
        pltpu.sync_copy(
            w_hbm.at[pl.ds(wid * nmeta, nmeta)], w_all.at[pl.ds(0, nmeta)]
        )

        def q_copy(ci, q_v, sem):
            return pltpu.make_async_copy(
                q_hbm.at[pl.ds(wbase + ci * CHUNK, CHUNK)], q_v, sem
            )

        def g_copy(ci, g_v, k, sem):
            return pltpu.make_async_copy(
                v_hbm.at[idx_all.at[ci * gpc + k]],
                g_v.at[pl.ds(k * 128, 128)],
                sem,
            )

        def issue(ci, g_v, q_v, sem):
            q_copy(ci, q_v, sem).start()
            for k in range(gpc):
                g_copy(ci, g_v, k, sem).start()

        def drain(ci, g_v, q_v, sem):
            q_copy(ci, q_v, sem).wait()
            for k in range(gpc):
                g_copy(ci, g_v, k, sem).wait()

        issue(0, g0, q0, sem0)
        issue(1, g1, q1, sem1)

        def pair_body(i, carry):
            for sl_i, (g_v, q_v, sem) in enumerate(
                ((g0, q0, sem0), (g1, q1, sem1))
            ):
                ci = 2 * i + sl_i
                drain(ci, g_v, q_v, sem)

                def row_body(r, carry2):
                    wvec = w_all[pl.ds((ci * CHUNK + r) * TOPK, SC_LANES)]
                    w0, w1, w2, w3 = wvec[0], wvec[1], wvec[2], wvec[3]
                    rb = r * TOPK
                    for sgm in range(d // SC_LANES):
                        sl = pl.ds(sgm * SC_LANES, SC_LANES)
                        acc = w0 * g_v[rb, sl]
                        acc = acc + w1 * g_v[rb + 1, sl]
                        acc = acc + w2 * g_v[rb + 2, sl]
                        acc = acc + w3 * g_v[rb + 3, sl]
                        o_v[r, sl] = acc * q_v[r, sl]
                    return carry2

                lax.fori_loop(0, CHUNK, row_body, 0)
                pltpu.sync_copy(
                    o_v, out_hbm.at[pl.ds(wbase + ci * CHUNK, CHUNK)]
                )

                @pl.when(ci + 2 < nchunk)
                def _():
                    issue(ci + 2, g_v, q_v, sem)

            return carry

        lax.fori_loop(0, nchunk // 2, pair_body, 0)

    return sc_k(v_flat, q_flat, w2, i2)


def kernel(x, W_qkv, b_qkv):
    batch, c, hh, ww = x.shape
    n = hh * ww
    h, d = NUM_HEADS, HEAD_DIM
    xt = jnp.transpose(x.reshape(batch, c, n), (0, 2, 1))
    wt = jnp.transpose(W_qkv.reshape(c, 3, h, d), (1, 2, 0, 3))
    br = b_qkv.reshape(3, h, d)[:, :, None, :]

    outs = []
    rows = h * n
    for b in range(batch):
        q4, v4, w4, i4 = _tc_stage(xt[b : b + 1], wt, br, 1)
        out_p = _sc_combine(
            v4.reshape(rows, VPAD),
            q4.reshape(rows, VPAD),
            w4.reshape(rows * TOPK),
            i4.reshape(rows * TOPK // 128, 128),
            1,
        )
        out_b = out_p.reshape(1, h, n, VPAD)[..., :d]
        outs.append(
            jnp.transpose(out_b, (0, 2, 1, 3)).reshape(1, c, hh, ww)
        )
    return jnp.concatenate(outs, axis=0)
